# baseline (speedup 1.0000x reference)
import functools

import jax
import jax.numpy as jnp
from jax import lax
from jax.experimental import pallas as pl
from jax.experimental.pallas import tpu as pltpu
from jax.experimental.pallas import tpu_sc as plsc

_NC = 2
_NS = 16
_NW = _NC * _NS
_L = 16


@functools.lru_cache(maxsize=None)
def _build(B0, S, V, D):
  bw = B0 // _NW
  NB = bw // _L

  mesh = plsc.VectorSubcoreMesh(core_axis_name="c", subcore_axis_name="s")

  @functools.partial(
      pl.kernel,
      mesh=mesh,
      out_type=jax.ShapeDtypeStruct((S, D, B0), jnp.float32),
      compiler_params=pltpu.CompilerParams(
          use_tc_tiling_on_sc=False, needs_layout_passes=False),
      scratch_types=[
          pltpu.VMEM((bw * S,), jnp.int32),
          pltpu.VMEM((bw,), jnp.int32),
          pltpu.VMEM((bw,), jnp.int32),
          pltpu.VMEM((2 * bw, D), jnp.float32),
          pltpu.VMEM((2 * D, bw), jnp.float32),
          pltpu.SemaphoreType.DMA,
          pltpu.SemaphoreType.DMA,
          pltpu.SemaphoreType.DMA,
          pltpu.SemaphoreType.DMA,
      ],
  )
  def k(xf_hbm, table_hbm, out_hbm, xslab, idx0, idx1, gbuf, obuf,
        gsem0, gsem1, wsem0, wsem1):
    w = lax.axis_index("s") * _NC + lax.axis_index("c")
    b0 = w * bw
    lanes = lax.iota(jnp.int32, _L)

    pltpu.sync_copy(xf_hbm.at[pl.ds(b0 * S, bw * S)], xslab)

    def start_gather(s):
      par = lax.rem(s, 2)

      @pl.when(par == 0)
      def _():
        @plsc.parallel_loop(0, bw // _L)
        def _x0(l):
          col = plsc.load_gather(
              xslab, [(l * _L + lanes) * S + s])
          plsc.store_scatter(idx0, [l * _L + lanes], col)

        pltpu.async_copy(table_hbm.at[idx0], gbuf.at[pl.ds(0, bw)], gsem0)

      @pl.when(par == 1)
      def _():
        @plsc.parallel_loop(0, bw // _L)
        def _x1(l):
          col = plsc.load_gather(
              xslab, [(l * _L + lanes) * S + s])
          plsc.store_scatter(idx1, [l * _L + lanes], col)

        pltpu.async_copy(table_hbm.at[idx1], gbuf.at[pl.ds(bw, bw)], gsem1)

    def wait_gather(par):
      @pl.when(par == 0)
      def _():
        pltpu.make_async_copy(
            table_hbm.at[idx0], gbuf.at[pl.ds(0, bw)], gsem0).wait()

      @pl.when(par == 1)
      def _():
        pltpu.make_async_copy(
            table_hbm.at[idx1], gbuf.at[pl.ds(bw, bw)], gsem1).wait()

    def out_slab(s):
      return out_hbm.at[s, pl.ds(0, D), pl.ds(b0, bw)]

    def start_write(s, par):
      @pl.when(par == 0)
      def _():
        pltpu.async_copy(obuf.at[pl.ds(0, D)], out_slab(s), wsem0)

      @pl.when(par == 1)
      def _():
        pltpu.async_copy(obuf.at[pl.ds(D, D)], out_slab(s), wsem1)

    def wait_write(s, par):
      @pl.when(par == 0)
      def _():
        pltpu.make_async_copy(obuf.at[pl.ds(0, D)], out_slab(s), wsem0).wait()

      @pl.when(par == 1)
      def _():
        pltpu.make_async_copy(obuf.at[pl.ds(D, D)], out_slab(s), wsem1).wait()

    start_gather(0)

    def step(s, _):
      par = lax.rem(s, 2)

      @pl.when(s + 1 < S)
      def _():
        start_gather(s + 1)

      @pl.when(s >= 2)
      def _():
        wait_write(s - 2, par)

      wait_gather(par)

      grow0 = par * bw
      orow0 = par * D

      @plsc.parallel_loop(0, NB * D, unroll=16)
      def _tr(i):
        vb = i // D
        e = i - vb * D
        vv = grow0 + vb * _L + lanes
        row = plsc.load_gather(gbuf, [vv, jnp.full((_L,), e, jnp.int32)])
        plsc.store_scatter(
            obuf, [jnp.full((_L,), orow0 + e, jnp.int32),
                   vb * _L + lanes], row)
      start_write(s, par)
      return 0

    lax.fori_loop(0, S, step, 0)
    wait_write(S - 2, lax.rem(S - 2, 2))
    wait_write(S - 1, lax.rem(S - 1, 2))

  return k


def kernel(x, table):
  B0, S = x.shape
  V, D = table.shape
  out2 = _build(B0, S, V, D)(x.reshape(B0 * S).astype(jnp.int32), table)
  return out2.transpose(2, 0, 1)


# speedup vs baseline: 1.0720x; 1.0720x over previous
import functools

import jax
import jax.numpy as jnp
from jax import lax
from jax.experimental import pallas as pl
from jax.experimental.pallas import tpu as pltpu
from jax.experimental.pallas import tpu_sc as plsc

_NC = 2
_NS = 16
_NW = _NC * _NS
_L = 16


@functools.lru_cache(maxsize=None)
def _build(B0, S, V, D):
  bw = B0 // _NW
  NB = bw // _L

  mesh = plsc.VectorSubcoreMesh(core_axis_name="c", subcore_axis_name="s")

  @functools.partial(
      pl.kernel,
      mesh=mesh,
      out_type=jax.ShapeDtypeStruct((S, D, B0), jnp.float32),
      compiler_params=pltpu.CompilerParams(
          use_tc_tiling_on_sc=False, needs_layout_passes=False),
      scratch_types=[
          pltpu.VMEM((bw * S,), jnp.int32),
          pltpu.VMEM((bw,), jnp.int32),
          pltpu.VMEM((bw,), jnp.int32),
          pltpu.VMEM((2 * bw, D), jnp.float32),
          pltpu.VMEM((2 * D, bw), jnp.float32),
          pltpu.SemaphoreType.DMA,
          pltpu.SemaphoreType.DMA,
          pltpu.SemaphoreType.DMA,
          pltpu.SemaphoreType.DMA,
      ],
  )
  def k(xf_hbm, table_hbm, out_hbm, xslab, idx0, idx1, gbuf, obuf,
        gsem0, gsem1, wsem0, wsem1):
    w = lax.axis_index("s") * _NC + lax.axis_index("c")
    b0 = w * bw
    lanes = lax.iota(jnp.int32, _L)

    pltpu.sync_copy(xf_hbm.at[pl.ds(b0 * S, bw * S)], xslab)

    def start_gather(s):
      par = lax.rem(s, 2)

      @pl.when(par == 0)
      def _():
        @plsc.parallel_loop(0, bw // _L)
        def _x0(l):
          col = plsc.load_gather(
              xslab, [(l * _L + lanes) * S + s])
          plsc.store_scatter(idx0, [l * _L + lanes], col)

        pltpu.async_copy(table_hbm.at[idx0], gbuf.at[pl.ds(0, bw)], gsem0)

      @pl.when(par == 1)
      def _():
        @plsc.parallel_loop(0, bw // _L)
        def _x1(l):
          col = plsc.load_gather(
              xslab, [(l * _L + lanes) * S + s])
          plsc.store_scatter(idx1, [l * _L + lanes], col)

        pltpu.async_copy(table_hbm.at[idx1], gbuf.at[pl.ds(bw, bw)], gsem1)

    def wait_gather(par):
      @pl.when(par == 0)
      def _():
        pltpu.make_async_copy(
            table_hbm.at[idx0], gbuf.at[pl.ds(0, bw)], gsem0).wait()

      @pl.when(par == 1)
      def _():
        pltpu.make_async_copy(
            table_hbm.at[idx1], gbuf.at[pl.ds(bw, bw)], gsem1).wait()

    def out_slab(s):
      return out_hbm.at[s, pl.ds(0, D), pl.ds(b0, bw)]

    def start_write(s, par):
      @pl.when(par == 0)
      def _():
        pltpu.async_copy(obuf.at[pl.ds(0, D)], out_slab(s), wsem0)

      @pl.when(par == 1)
      def _():
        pltpu.async_copy(obuf.at[pl.ds(D, D)], out_slab(s), wsem1)

    def wait_write(s, par):
      @pl.when(par == 0)
      def _():
        pltpu.make_async_copy(obuf.at[pl.ds(0, D)], out_slab(s), wsem0).wait()

      @pl.when(par == 1)
      def _():
        pltpu.make_async_copy(obuf.at[pl.ds(D, D)], out_slab(s), wsem1).wait()

    start_gather(0)

    def step(s, _):
      par = lax.rem(s, 2)

      @pl.when(s + 1 < S)
      def _():
        start_gather(s + 1)

      @pl.when(s >= 2)
      def _():
        wait_write(s - 2, par)

      wait_gather(par)

      grow0 = par * bw
      orow0 = par * D

      @plsc.parallel_loop(0, NB * D, unroll=32)
      def _tr(i):
        vb = i // D
        e = i - vb * D
        vv = grow0 + vb * _L + lanes
        row = plsc.load_gather(gbuf, [vv, jnp.full((_L,), e, jnp.int32)])
        plsc.store_scatter(
            obuf, [jnp.full((_L,), orow0 + e, jnp.int32),
                   vb * _L + lanes], row)
      start_write(s, par)
      return 0

    lax.fori_loop(0, S, step, 0)
    wait_write(S - 2, lax.rem(S - 2, 2))
    wait_write(S - 1, lax.rem(S - 1, 2))

  return k


def kernel(x, table):
  B0, S = x.shape
  V, D = table.shape
  out2 = _build(B0, S, V, D)(x.reshape(B0 * S).astype(jnp.int32), table)
  return out2.transpose(2, 0, 1)
